# baseline (speedup 1.0000x reference)
import functools

import jax
import jax.numpy as jnp
from jax import lax
from jax.experimental import pallas as pl
from jax.experimental.pallas import tpu as pltpu
from jax.experimental.pallas import tpu_sc as plsc

N_NODES = 10000
N_EDGES = 320000
D_FEAT = 128

NC = 2
NS = 16
NW = NC * NS
CHUNK = 128
N_CHUNKS = 80
HALF = N_CHUNKS // 2
E_PER_W = N_CHUNKS * CHUNK
E_PAD = NW * E_PER_W
N_PAD = 10240
ROWS_PER_TILE = N_PAD // NS


def _sc_partials(x, row_idx, col_idx, zeros):
    mesh = plsc.VectorSubcoreMesh(core_axis_name="c", subcore_axis_name="s")

    @functools.partial(
        pl.kernel,
        mesh=mesh,
        out_type=jax.ShapeDtypeStruct((NC, N_PAD, D_FEAT), jnp.float32),
        scratch_types=[
            pltpu.VMEM((HALF, CHUNK), jnp.int32),
            pltpu.VMEM((HALF, CHUNK), jnp.int32),
            pltpu.VMEM((CHUNK, D_FEAT), jnp.float32),
            pltpu.VMEM((CHUNK, D_FEAT), jnp.float32),
            pltpu.VMEM_SHARED((N_PAD, D_FEAT), jnp.float32),
            pltpu.SemaphoreType.DMA,
            pltpu.SemaphoreType.DMA,
            pltpu.SemaphoreType.DMA,
            pltpu.SemaphoreType.DMA,
        ],
    )
    def k(x_hbm, row_hbm, col_hbm, zero_hbm, out_hbm,
          row_v, col_v, msg_a, msg_b, acc, sem_a, sem_b, sem_sa, sem_sb):
        c = lax.axis_index("c")
        s = lax.axis_index("s")
        wid = s * NC + c
        r0 = s * ROWS_PER_TILE
        pltpu.sync_copy(zero_hbm.at[pl.ds(r0, ROWS_PER_TILE)],
                        acc.at[pl.ds(r0, ROWS_PER_TILE)])
        plsc.subcore_barrier()

        for h in range(2):
            pltpu.sync_copy(row_hbm.at[wid, h], row_v)
            pltpu.sync_copy(col_hbm.at[wid, h], col_v)
            pltpu.async_copy(x_hbm.at[row_v.at[0]], msg_a, sem_a)
            pltpu.async_copy(x_hbm.at[row_v.at[1]], msg_b, sem_b)

            def body(i, carry):
                ja = 2 * i
                jb = 2 * i + 1
                pltpu.make_async_copy(x_hbm.at[row_v.at[ja]], msg_a, sem_a).wait()
                pltpu.async_copy(msg_a, acc.at[col_v.at[ja]], sem_sa, add=True)
                pltpu.make_async_copy(x_hbm.at[row_v.at[jb]], msg_b, sem_b).wait()
                pltpu.async_copy(msg_b, acc.at[col_v.at[jb]], sem_sb, add=True)

                @pl.when(ja + 2 < HALF)
                def _():
                    pltpu.make_async_copy(msg_a, acc.at[col_v.at[ja]], sem_sa).wait()
                    pltpu.async_copy(x_hbm.at[row_v.at[ja + 2]], msg_a, sem_a)
                    pltpu.make_async_copy(msg_b, acc.at[col_v.at[jb]], sem_sb).wait()
                    pltpu.async_copy(x_hbm.at[row_v.at[jb + 2]], msg_b, sem_b)

                return carry

            lax.fori_loop(0, HALF // 2, body, 0)
            pltpu.make_async_copy(msg_a, acc.at[col_v.at[HALF - 2]], sem_sa).wait()
            pltpu.make_async_copy(msg_b, acc.at[col_v.at[HALF - 1]], sem_sb).wait()

        plsc.subcore_barrier()
        pltpu.sync_copy(acc.at[pl.ds(r0, ROWS_PER_TILE)],
                        out_hbm.at[c, pl.ds(r0, ROWS_PER_TILE)])

    return k(x, row_idx, col_idx, zeros)


def _tc_add(partials):
    blk = 2000

    def body(p_ref, o_ref):
        o_ref[...] = p_ref[0] + p_ref[1]

    return pl.pallas_call(
        body,
        grid=(N_NODES // blk,),
        in_specs=[pl.BlockSpec((NC, blk, D_FEAT), lambda i: (0, i, 0))],
        out_specs=pl.BlockSpec((blk, D_FEAT), lambda i: (i, 0)),
        out_shape=jax.ShapeDtypeStruct((N_NODES, D_FEAT), jnp.float32),
    )(partials)


def kernel(graph_or_x, edge_index):
    x = graph_or_x.astype(jnp.float32)
    ei = edge_index.astype(jnp.int32)
    n_extra = E_PAD - N_EDGES
    pad = jnp.arange(n_extra, dtype=jnp.int32)
    row = jnp.concatenate([ei[0], pad % N_NODES])
    col = jnp.concatenate([ei[1], N_NODES + pad % (N_PAD - N_NODES)])
    row = row.reshape(NW, 2, HALF, CHUNK)
    col = col.reshape(NW, 2, HALF, CHUNK)
    zeros = jnp.zeros((N_PAD, D_FEAT), jnp.float32)
    partials = _sc_partials(x, row, col, zeros)
    return _tc_add(partials)


# speedup vs baseline: 1.2889x; 1.2889x over previous
import functools

import jax
import jax.numpy as jnp
from jax import lax
from jax.experimental import pallas as pl
from jax.experimental.pallas import tpu as pltpu
from jax.experimental.pallas import tpu_sc as plsc

N_NODES = 10000
N_EDGES = 320000
D_FEAT = 128

NC = 2
NS = 16
NW = NC * NS
CHUNK = 128
N_CHUNKS = 80
HALF = N_CHUNKS // 2
E_PER_W = N_CHUNKS * CHUNK
E_PAD = NW * E_PER_W
N_PAD = 10240
ROWS_PER_TILE = N_PAD // NS


def _sc_partials(x, row_idx, col_idx):
    mesh = plsc.VectorSubcoreMesh(core_axis_name="c", subcore_axis_name="s")

    @functools.partial(
        pl.kernel,
        mesh=mesh,
        out_type=jax.ShapeDtypeStruct((NC, N_PAD, D_FEAT), jnp.float32),
        scratch_types=[
            pltpu.VMEM((HALF, CHUNK), jnp.int32),
            pltpu.VMEM((HALF, CHUNK), jnp.int32),
            pltpu.VMEM((CHUNK, D_FEAT), jnp.float32),
            pltpu.VMEM((CHUNK, D_FEAT), jnp.float32),
            pltpu.VMEM_SHARED((N_PAD, D_FEAT), jnp.float32),
            pltpu.SemaphoreType.DMA,
            pltpu.SemaphoreType.DMA,
            pltpu.SemaphoreType.DMA,
            pltpu.SemaphoreType.DMA,
        ],
    )
    def k(x_hbm, row_hbm, col_hbm, out_hbm,
          row_v, col_v, msg_a, msg_b, acc, sem_a, sem_b, sem_sa, sem_sb):
        c = lax.axis_index("c")
        s = lax.axis_index("s")
        wid = s * NC + c
        r0 = s * ROWS_PER_TILE
        zvec = jnp.zeros((16,), jnp.float32)

        def zbody(i, carry):
            for l in range(D_FEAT // 16):
                msg_a[i, pl.ds(l * 16, 16)] = zvec
            return carry

        lax.fori_loop(0, CHUNK, zbody, 0)
        for t in range(ROWS_PER_TILE // CHUNK):
            pltpu.sync_copy(msg_a, acc.at[pl.ds(r0 + t * CHUNK, CHUNK)])
        plsc.subcore_barrier()

        for h in range(2):
            pltpu.sync_copy(row_hbm.at[wid, h], row_v)
            pltpu.sync_copy(col_hbm.at[wid, h], col_v)
            pltpu.async_copy(x_hbm.at[row_v.at[0]], msg_a, sem_a)

            def body(i, carry):
                ja = 2 * i
                jb = 2 * i + 1
                pltpu.async_copy(x_hbm.at[row_v.at[jb]], msg_b, sem_b)
                pltpu.make_async_copy(x_hbm.at[row_v.at[ja]], msg_a, sem_a).wait()
                pltpu.sync_copy(msg_a, acc.at[col_v.at[ja]], add=True)

                @pl.when(jb + 1 < HALF)
                def _():
                    pltpu.async_copy(x_hbm.at[row_v.at[jb + 1]], msg_a, sem_a)

                pltpu.make_async_copy(x_hbm.at[row_v.at[jb]], msg_b, sem_b).wait()
                pltpu.sync_copy(msg_b, acc.at[col_v.at[jb]], add=True)
                return carry

            lax.fori_loop(0, HALF // 2, body, 0)

        plsc.subcore_barrier()
        pltpu.sync_copy(acc.at[pl.ds(r0, ROWS_PER_TILE)],
                        out_hbm.at[c, pl.ds(r0, ROWS_PER_TILE)])

    return k(x, row_idx, col_idx)


def _tc_add(partials):
    blk = 2000

    def body(p_ref, o_ref):
        o_ref[...] = p_ref[0] + p_ref[1]

    return pl.pallas_call(
        body,
        grid=(N_NODES // blk,),
        in_specs=[pl.BlockSpec((NC, blk, D_FEAT), lambda i: (0, i, 0))],
        out_specs=pl.BlockSpec((blk, D_FEAT), lambda i: (i, 0)),
        out_shape=jax.ShapeDtypeStruct((N_NODES, D_FEAT), jnp.float32),
    )(partials)


def kernel(graph_or_x, edge_index):
    x = graph_or_x.astype(jnp.float32)
    ei = edge_index.astype(jnp.int32)
    n_extra = E_PAD - N_EDGES
    pad = jnp.arange(n_extra, dtype=jnp.int32)
    row = jnp.concatenate([ei[0], pad % N_NODES])
    col = jnp.concatenate([ei[1], N_NODES + pad % (N_PAD - N_NODES)])
    row = row.reshape(NW, 2, HALF, CHUNK)
    col = col.reshape(NW, 2, HALF, CHUNK)
    partials = _sc_partials(x, row, col)
    return _tc_add(partials)
